# baseline (speedup 1.0000x reference)
import functools

import jax
import jax.numpy as jnp
from jax import lax
from jax.experimental import pallas as pl
from jax.experimental.pallas import tpu as pltpu
from jax.experimental.pallas import tpu_sc as plsc

N = 10000
K = 32
C_IN = 128
C_NB = 32
UP = 4
C_OUT = C_IN * UP
CC = C_IN + C_NB

NC, NS = 2, 16
NW = NC * NS
CH = 128
PER_W = 10240
NCHUNK = PER_W // CH
NKP = NW * PER_W
NP = NKP // K
PPT = NP // NW
CPD = CH // K

BN = 256
GRID = NP // BN


def _mesh():
    return plsc.VectorSubcoreMesh(core_axis_name="c", subcore_axis_name="s",
                                  num_cores=NC, num_subcores=NS)


NBUF = 4


@functools.cache
def _sc_meta_gather_kernel():
    @functools.partial(
        pl.kernel,
        out_type=jax.ShapeDtypeStruct((NKP, 16), jnp.float32),
        mesh=_mesh(),
        scratch_types=[
            pltpu.VMEM((NCHUNK, CH), jnp.int32),
            pltpu.VMEM((NBUF, CH, 16), jnp.float32),
        ] + [pltpu.SemaphoreType.DMA] * (2 * NBUF),
        compiler_params=pltpu.CompilerParams(use_tc_tiling_on_sc=False),
    )
    def sc_meta_gather(meta_hbm, idx_hbm, out_hbm, idx_v, buf_v, *sems):
        semg = sems[:NBUF]
        semw = sems[NBUF:]
        wid = lax.axis_index("c") * NS + lax.axis_index("s")
        base_w = wid * PER_W

        pltpu.sync_copy(idx_hbm.at[pl.ds(wid * NCHUNK, NCHUNK)], idx_v)

        def g_start(g, b):
            pltpu.async_copy(meta_hbm.at[idx_v.at[g]], buf_v.at[b], semg[b])

        def g_wait(g, b):
            pltpu.make_async_copy(meta_hbm.at[idx_v.at[g]], buf_v.at[b],
                                  semg[b]).wait()

        def w_start(g, b):
            pltpu.async_copy(buf_v.at[b],
                             out_hbm.at[pl.ds(base_w + g * CH, CH)], semw[b])

        def w_wait(g, b):
            pltpu.make_async_copy(buf_v.at[b],
                                  out_hbm.at[pl.ds(base_w + g * CH, CH)],
                                  semw[b]).wait()

        for b in range(NBUF - 1):
            g_start(b, b)

        def outer(j, carry):
            for b in range(NBUF):
                g = j * NBUF + b

                @pl.when(g >= 1)
                def _():
                    w_wait(g - 1, (b - 1) % NBUF)

                @pl.when(g + NBUF - 1 < NCHUNK)
                def _():
                    g_start(g + NBUF - 1, (b - 1) % NBUF)

                g_wait(g, b)
                w_start(g, b)
            return carry

        lax.fori_loop(0, NCHUNK // NBUF, outer, 0)
        w_wait(NCHUNK - 1, (NCHUNK - 1) % NBUF)

    return sc_meta_gather


@functools.cache
def _sc_wreduce_kernel(nph):
    ppt = nph // NW
    nchunk = ppt // CPD

    @functools.partial(
        pl.kernel,
        out_type=jax.ShapeDtypeStruct((nph, C_IN), jnp.float32),
        mesh=_mesh(),
        scratch_types=[
            pltpu.VMEM((nchunk, CH), jnp.int32),
            pltpu.VMEM((ppt, K), jnp.float32),
            pltpu.VMEM((NBUF, CH, C_IN // 2), jnp.int32),
            pltpu.VMEM((2, 2 * CPD, C_IN), jnp.float32),
        ] + [pltpu.SemaphoreType.DMA] * (NBUF + 2),
        compiler_params=pltpu.CompilerParams(use_tc_tiling_on_sc=False),
    )
    def sc_wreduce(feat_hbm, idx_hbm, p_hbm, out_hbm, idx_v, p_v, rows_v,
                   out_v, *sems):
        NCHUNK = nchunk
        PPT = ppt
        semg = sems[:NBUF]
        semw = sems[NBUF:]
        wid = lax.axis_index("c") * NS + lax.axis_index("s")

        pltpu.sync_copy(idx_hbm.at[pl.ds(wid * NCHUNK, NCHUNK)], idx_v)
        pltpu.sync_copy(p_hbm.at[pl.ds(wid * PPT, PPT)], p_v)

        def g_start(g, b):
            pltpu.async_copy(feat_hbm.at[idx_v.at[g]], rows_v.at[b], semg[b])

        def g_wait(g, b):
            pltpu.make_async_copy(feat_hbm.at[idx_v.at[g]], rows_v.at[b],
                                  semg[b]).wait()

        def w_start(pair, os):
            base = wid * PPT + pair * 2 * CPD
            pltpu.async_copy(out_v.at[os],
                             out_hbm.at[pl.ds(base, 2 * CPD)], semw[os])

        def w_wait(pair, os):
            base = wid * PPT + pair * 2 * CPD
            pltpu.make_async_copy(out_v.at[os],
                                  out_hbm.at[pl.ds(base, 2 * CPD)],
                                  semw[os]).wait()

        def compute(g, b, half, os):
            rows = rows_v.at[b]
            for ii in range(CPD):
                pp = g * CPD + ii
                pv = [p_v[pp, pl.ds(h * 16, 16)] for h in range(K // 16)]
                acc = [None] * (C_IN // 16)
                for k in range(K):
                    pk = pv[k // 16][k % 16]
                    for j in range(C_IN // 32):
                        v = rows[ii * K + k, pl.ds(j * 16, 16)]
                        a = lax.bitcast_convert_type(v << 16, jnp.float32)
                        b2 = lax.bitcast_convert_type(
                            v & jnp.int32(-65536), jnp.float32)
                        ta = a * pk
                        tb = b2 * pk
                        ca, cb = 2 * j, 2 * j + 1
                        acc[ca] = ta if acc[ca] is None else acc[ca] + ta
                        acc[cb] = tb if acc[cb] is None else acc[cb] + tb
                for c in range(C_IN // 16):
                    out_v[os, half * CPD + ii, pl.ds(c * 16, 16)] = acc[c]

        for b in range(NBUF - 1):
            g_start(b, b)

        def outer(j, carry):
            for b in range(NBUF):
                g = j * NBUF + b
                half = b % 2
                os = (b // 2) % 2

                @pl.when(g + NBUF - 1 < NCHUNK)
                def _():
                    g_start(g + NBUF - 1, (b - 1) % NBUF)

                g_wait(g, b)

                if half == 0:
                    @pl.when(g // 2 >= 2)
                    def _():
                        w_wait(g // 2 - 2, os)

                compute(g, b, half, os)

                if half == 1:
                    w_start(g // 2, os)
            return carry

        lax.fori_loop(0, NCHUNK // NBUF, outer, 0)
        w_wait(NCHUNK // 2 - 2, 0)
        w_wait(NCHUNK // 2 - 1, 1)

    return sc_wreduce


BA = 512


def _tc_meta_body(xyzp_ref, feat_ref, wattf_ref, meta_ref):
    s = feat_ref[...] @ wattf_ref[...].T
    xyz3 = xyzp_ref[:, 0:3]
    z = jnp.zeros((BA, 12), jnp.float32)
    meta_ref[...] = jnp.concatenate([xyz3, s, z], axis=1)


def _tc_meta(xyz_pad, feat_pad, wattf):
    return pl.pallas_call(
        _tc_meta_body,
        grid=(NP // BA,),
        in_specs=[
            pl.BlockSpec((BA, 16), lambda i: (i, 0)),
            pl.BlockSpec((BA, C_IN), lambda i: (i, 0)),
            pl.BlockSpec((1, C_IN), lambda i: (0, 0)),
        ],
        out_specs=pl.BlockSpec((BA, 16), lambda i: (i, 0)),
        out_shape=jax.ShapeDtypeStruct((NP, 16), jnp.float32),
    )(xyz_pad, feat_pad, wattf)


def _tc_attn_body(xyzp_ref, mg_ref, W9_ref, w9_ref, bnb_ref, ones3_ref,
                  wattnsq_ref, p_ref, aggn_ref):
    mg2 = mg_ref[...]
    c16 = jnp.broadcast_to(xyzp_ref[...][:, None, :],
                           (BN, K, 16)).reshape(BN * K, 16)
    c3 = c16[:, 0:3]
    nbx = mg2[:, 0:3]
    s1 = mg2[:, 3:4]
    rel = c3 - nbx
    raw9 = jnp.concatenate([c3, nbx, rel], axis=1)
    d2b = (rel * rel) @ ones3_ref[...]
    distb = jnp.sqrt(d2b)
    nb2 = raw9 @ W9_ref[...] + distb * w9_ref[...] + bnb_ref[...]
    nb2 = jnp.where(nb2 >= 0, nb2, 0.2 * nb2)

    logitb = nb2 @ wattnsq_ref[...] + s1
    L3 = logitb.reshape(BN, K, C_NB)
    m = jnp.max(L3, axis=1, keepdims=True)
    e = jnp.exp(L3 - m)
    p3 = e * (1.0 / jnp.sum(e, axis=1, keepdims=True))

    nb3 = nb2.reshape(BN, K, C_NB)
    aggn_ref[...] = jnp.sum(p3 * nb3, axis=1)
    p_ref[...] = p3[:, :, 0]


def _tc_attn(xyz_pad, meta_g, W9, w9, bnb, ones3, wattnsq):
    nph = xyz_pad.shape[0]
    return pl.pallas_call(
        _tc_attn_body,
        grid=(nph // BN,),
        in_specs=[
            pl.BlockSpec((BN, 16), lambda i: (i, 0)),
            pl.BlockSpec((BN * K, 16), lambda i: (i, 0)),
            pl.BlockSpec((9, C_NB), lambda i: (0, 0)),
            pl.BlockSpec((1, C_NB), lambda i: (0, 0)),
            pl.BlockSpec((1, C_NB), lambda i: (0, 0)),
            pl.BlockSpec((3, C_NB), lambda i: (0, 0)),
            pl.BlockSpec((C_NB, C_NB), lambda i: (0, 0)),
        ],
        out_specs=[
            pl.BlockSpec((BN, K), lambda i: (i, 0)),
            pl.BlockSpec((BN, C_NB), lambda i: (i, 0)),
        ],
        out_shape=[
            jax.ShapeDtypeStruct((nph, K), jnp.float32),
            jax.ShapeDtypeStruct((nph, C_NB), jnp.float32),
        ],
    )(xyz_pad, meta_g, W9, w9, bnb, ones3, wattnsq)


def _tc_final_body(xyzp_ref, feat_ref, aggf_ref, aggn_ref,
                   Wof_ref, Won_ref, bout_ref, Wsc_ref, bsc_ref,
                   Wco_ref, bco_ref, fea_ref, pred_ref):
    out = (aggf_ref[...] @ Wof_ref[...] + aggn_ref[...] @ Won_ref[...]
           + bout_ref[...])
    shortcut = feat_ref[...] @ Wsc_ref[...] + bsc_ref[...]
    fea = out + shortcut
    fea = jnp.where(fea >= 0, fea, 0.2 * fea)
    pieces = [fea[:, u * C_IN:(u + 1) * C_IN] for u in range(UP)]
    fea_ref[...] = jnp.stack(pieces, axis=1).reshape(BN * UP, C_IN)

    xyzc = xyzp_ref[:, 0:3]
    Wco = Wco_ref[...]
    bco = bco_ref[...]
    offs = []
    for u in range(UP):
        off = pieces[u] @ Wco + bco
        offs.append(off + xyzc)
    pred_ref[...] = jnp.concatenate(offs, axis=1)


def _tc_final(xyz_pad, feat_pad, agg_f, agg_n, Wof, Won, bout, Wsc, bsc,
              Wco, bco):
    full = lambda shape: pl.BlockSpec(shape, lambda i: tuple(0 for _ in shape))
    return pl.pallas_call(
        _tc_final_body,
        grid=(GRID,),
        in_specs=[
            pl.BlockSpec((BN, 16), lambda i: (i, 0)),
            pl.BlockSpec((BN, C_IN), lambda i: (i, 0)),
            pl.BlockSpec((BN, C_IN), lambda i: (i, 0)),
            pl.BlockSpec((BN, C_NB), lambda i: (i, 0)),
            full((C_IN, C_OUT)),
            full((C_NB, C_OUT)),
            full((1, C_OUT)),
            full((C_IN, C_OUT)),
            full((1, C_OUT)),
            full((C_IN, 3)),
            full((1, 3)),
        ],
        out_specs=[
            pl.BlockSpec((BN * UP, C_IN), lambda i: (i, 0)),
            pl.BlockSpec((BN, 12), lambda i: (i, 0)),
        ],
        out_shape=[
            jax.ShapeDtypeStruct((NP * UP, C_IN), jnp.float32),
            jax.ShapeDtypeStruct((NP, 12), jnp.float32),
        ],
    )(xyz_pad, feat_pad, agg_f, agg_n, Wof, Won, bout, Wsc, bsc, Wco, bco)


def _sc_meta_gather(meta, idx2):
    return _sc_meta_gather_kernel()(meta, idx2)


def _sc_wreduce(feat_pad, idx2, p):
    return _sc_wreduce_kernel(p.shape[0])(feat_pad, idx2, p)


def kernel(xyz, feature, neighbors_idx, W_nb, b_nb, w_att, W_out, b_out,
           W_sc, b_sc, W_coord, b_coord):
    xyzf = xyz.reshape(N, 3)
    featf = feature.reshape(N, C_IN)
    idx = neighbors_idx.reshape(N * K).astype(jnp.int32)

    xyz_pad = jnp.pad(xyzf, ((0, NP - N), (0, 13)))
    feat_pad = jnp.pad(featf, ((0, NP - N), (0, 0)))
    idx2 = jnp.pad(idx, (0, NKP - N * K)).reshape(NKP // CH, CH)

    featbf = jnp.pad(featf.astype(jnp.bfloat16), ((0, NP - N), (0, 0)))
    featpk = lax.bitcast_convert_type(
        featbf.reshape(NP, C_IN // 2, 2), jnp.int32)

    wattf = w_att[:C_IN].reshape(1, C_IN)
    wattn = w_att[C_IN:].reshape(1, C_NB)
    perm = [32 * (m // 2) + 2 * t + (m % 2)
            for m in range(C_IN // 16) for t in range(16)]
    Wof = W_out[:C_IN][jnp.array(perm, jnp.int32)]
    Won = W_out[C_IN:]

    W9 = W_nb[0:9]
    w9 = W_nb[9:10]
    ones3 = jnp.ones((3, C_NB), jnp.float32)
    wattnsq = jnp.broadcast_to(w_att[C_IN:].reshape(C_NB, 1), (C_NB, C_NB))

    meta = _tc_meta(xyz_pad, feat_pad, wattf)
    meta_g = _sc_meta_gather(meta, idx2)

    nph = NP // 2
    nrh = nph * K // CH
    aggf_h, aggn_h = [], []
    for h in range(2):
        psl = slice(h * nph, (h + 1) * nph)
        p_h, an_h = _tc_attn(xyz_pad[psl], meta_g[h * nph * K:(h + 1) * nph * K],
                             W9, w9, b_nb.reshape(1, C_NB), ones3, wattnsq)
        af_h = _sc_wreduce(featpk, idx2[h * nrh:(h + 1) * nrh], p_h)
        aggf_h.append(af_h)
        aggn_h.append(an_h)
    agg_f = jnp.concatenate(aggf_h, axis=0)
    agg_n = jnp.concatenate(aggn_h, axis=0)

    fea4, pred = _tc_final(
        xyz_pad, feat_pad, agg_f, agg_n,
        Wof, Won, b_out.reshape(1, C_OUT),
        W_sc, b_sc.reshape(1, C_OUT),
        W_coord, b_coord.reshape(1, 3),
    )

    pred_coord = pred[:N].reshape(1, N * UP, 3)
    fea_out = fea4[:N * UP].reshape(1, N * UP, C_IN)
    return pred_coord, fea_out


# speedup vs baseline: 1.1128x; 1.1128x over previous
import functools

import jax
import jax.numpy as jnp
from jax import lax
from jax.experimental import pallas as pl
from jax.experimental.pallas import tpu as pltpu
from jax.experimental.pallas import tpu_sc as plsc

N = 10000
K = 32
C_IN = 128
C_NB = 32
UP = 4
C_OUT = C_IN * UP
CC = C_IN + C_NB

NC, NS = 2, 16
NW = NC * NS
CH = 128
PER_W = 10240
NCHUNK = PER_W // CH
NKP = NW * PER_W
NP = NKP // K
PPT = NP // NW
CPD = CH // K

BN = 256
GRID = NP // BN


def _mesh():
    return plsc.VectorSubcoreMesh(core_axis_name="c", subcore_axis_name="s",
                                  num_cores=NC, num_subcores=NS)


NBUF = 4


@functools.cache
def _sc_meta_gather_kernel():
    @functools.partial(
        pl.kernel,
        out_type=jax.ShapeDtypeStruct((NKP, 16), jnp.float32),
        mesh=_mesh(),
        scratch_types=[
            pltpu.VMEM((NCHUNK, CH), jnp.int32),
            pltpu.VMEM((NBUF, CH, 16), jnp.float32),
        ] + [pltpu.SemaphoreType.DMA] * (2 * NBUF),
        compiler_params=pltpu.CompilerParams(use_tc_tiling_on_sc=False),
    )
    def sc_meta_gather(meta_hbm, idx_hbm, out_hbm, idx_v, buf_v, *sems):
        semg = sems[:NBUF]
        semw = sems[NBUF:]
        wid = lax.axis_index("c") * NS + lax.axis_index("s")
        base_w = wid * PER_W

        pltpu.sync_copy(idx_hbm.at[pl.ds(wid * NCHUNK, NCHUNK)], idx_v)

        def g_start(g, b):
            pltpu.async_copy(meta_hbm.at[idx_v.at[g]], buf_v.at[b], semg[b])

        def g_wait(g, b):
            pltpu.make_async_copy(meta_hbm.at[idx_v.at[g]], buf_v.at[b],
                                  semg[b]).wait()

        def w_start(g, b):
            pltpu.async_copy(buf_v.at[b],
                             out_hbm.at[pl.ds(base_w + g * CH, CH)], semw[b])

        def w_wait(g, b):
            pltpu.make_async_copy(buf_v.at[b],
                                  out_hbm.at[pl.ds(base_w + g * CH, CH)],
                                  semw[b]).wait()

        for b in range(NBUF - 1):
            g_start(b, b)

        def outer(j, carry):
            for b in range(NBUF):
                g = j * NBUF + b

                @pl.when(g >= 1)
                def _():
                    w_wait(g - 1, (b - 1) % NBUF)

                @pl.when(g + NBUF - 1 < NCHUNK)
                def _():
                    g_start(g + NBUF - 1, (b - 1) % NBUF)

                g_wait(g, b)
                w_start(g, b)
            return carry

        lax.fori_loop(0, NCHUNK // NBUF, outer, 0)
        w_wait(NCHUNK - 1, (NCHUNK - 1) % NBUF)

    return sc_meta_gather


@functools.cache
def _sc_wreduce_kernel(nph):
    ppt = nph // NW
    nchunk = ppt // CPD

    @functools.partial(
        pl.kernel,
        out_type=jax.ShapeDtypeStruct((nph, C_IN), jnp.float32),
        mesh=_mesh(),
        scratch_types=[
            pltpu.VMEM((nchunk, CH), jnp.int32),
            pltpu.VMEM((ppt, K), jnp.float32),
            pltpu.VMEM((NBUF, CH, C_IN // 2), jnp.int32),
            pltpu.VMEM((2, 2 * CPD, C_IN), jnp.float32),
        ] + [pltpu.SemaphoreType.DMA] * (NBUF + 2),
        compiler_params=pltpu.CompilerParams(use_tc_tiling_on_sc=False),
    )
    def sc_wreduce(feat_hbm, idx_hbm, p_hbm, out_hbm, idx_v, p_v, rows_v,
                   out_v, *sems):
        NCHUNK = nchunk
        PPT = ppt
        semg = sems[:NBUF]
        semw = sems[NBUF:]
        wid = lax.axis_index("c") * NS + lax.axis_index("s")

        pltpu.sync_copy(idx_hbm.at[pl.ds(wid * NCHUNK, NCHUNK)], idx_v)
        pltpu.sync_copy(p_hbm.at[pl.ds(wid * PPT, PPT)], p_v)

        def g_start(g, b):
            pltpu.async_copy(feat_hbm.at[idx_v.at[g]], rows_v.at[b], semg[b])

        def g_wait(g, b):
            pltpu.make_async_copy(feat_hbm.at[idx_v.at[g]], rows_v.at[b],
                                  semg[b]).wait()

        def w_start(pair, os):
            base = wid * PPT + pair * 2 * CPD
            pltpu.async_copy(out_v.at[os],
                             out_hbm.at[pl.ds(base, 2 * CPD)], semw[os])

        def w_wait(pair, os):
            base = wid * PPT + pair * 2 * CPD
            pltpu.make_async_copy(out_v.at[os],
                                  out_hbm.at[pl.ds(base, 2 * CPD)],
                                  semw[os]).wait()

        def compute(g, b, half, os):
            rows = rows_v.at[b]
            for ii in range(CPD):
                pp = g * CPD + ii
                pv = [p_v[pp, pl.ds(h * 16, 16)] for h in range(K // 16)]
                acc = [None] * (C_IN // 16)
                for k in range(K):
                    pk = pv[k // 16][k % 16]
                    for j in range(C_IN // 32):
                        v = rows[ii * K + k, pl.ds(j * 16, 16)]
                        a = lax.bitcast_convert_type(v << 16, jnp.float32)
                        b2 = lax.bitcast_convert_type(
                            v & jnp.int32(-65536), jnp.float32)
                        ta = a * pk
                        tb = b2 * pk
                        ca, cb = 2 * j, 2 * j + 1
                        acc[ca] = ta if acc[ca] is None else acc[ca] + ta
                        acc[cb] = tb if acc[cb] is None else acc[cb] + tb
                for c in range(C_IN // 16):
                    out_v[os, half * CPD + ii, pl.ds(c * 16, 16)] = acc[c]

        for b in range(NBUF - 1):
            g_start(b, b)

        def outer(j, carry):
            for b in range(NBUF):
                g = j * NBUF + b
                half = b % 2
                os = (b // 2) % 2

                @pl.when(g + NBUF - 1 < NCHUNK)
                def _():
                    g_start(g + NBUF - 1, (b - 1) % NBUF)

                g_wait(g, b)

                if half == 0:
                    @pl.when(g // 2 >= 2)
                    def _():
                        w_wait(g // 2 - 2, os)

                compute(g, b, half, os)

                if half == 1:
                    w_start(g // 2, os)
            return carry

        lax.fori_loop(0, NCHUNK // NBUF, outer, 0)
        w_wait(NCHUNK // 2 - 2, 0)
        w_wait(NCHUNK // 2 - 1, 1)

    return sc_wreduce


BA = 512


def _tc_meta_body(xyzp_ref, feat_ref, wattf_ref, meta_ref):
    s = feat_ref[...] @ wattf_ref[...].T
    xyz3 = xyzp_ref[:, 0:3]
    z = jnp.zeros((BA, 12), jnp.float32)
    meta_ref[...] = jnp.concatenate([xyz3, s, z], axis=1)


def _tc_meta(xyz_pad, feat_pad, wattf):
    return pl.pallas_call(
        _tc_meta_body,
        grid=(NP // BA,),
        in_specs=[
            pl.BlockSpec((BA, 16), lambda i: (i, 0)),
            pl.BlockSpec((BA, C_IN), lambda i: (i, 0)),
            pl.BlockSpec((1, C_IN), lambda i: (0, 0)),
        ],
        out_specs=pl.BlockSpec((BA, 16), lambda i: (i, 0)),
        out_shape=jax.ShapeDtypeStruct((NP, 16), jnp.float32),
    )(xyz_pad, feat_pad, wattf)


def _tc_attn_body(xyzp_ref, mg_ref, W9_ref, w9_ref, bnb_ref, ones3_ref,
                  wattnsq_ref, p_ref, aggn_ref):
    mg2 = mg_ref[...]
    c16 = jnp.broadcast_to(xyzp_ref[...][:, None, :],
                           (BN, K, 16)).reshape(BN * K, 16)
    c3 = c16[:, 0:3]
    nbx = mg2[:, 0:3]
    s1 = mg2[:, 3:4]
    rel = c3 - nbx
    raw9 = jnp.concatenate([c3, nbx, rel], axis=1)
    d2b = (rel * rel) @ ones3_ref[...]
    distb = jnp.sqrt(d2b)
    nb2 = raw9 @ W9_ref[...] + distb * w9_ref[...] + bnb_ref[...]
    nb2 = jnp.where(nb2 >= 0, nb2, 0.2 * nb2)

    logitb = nb2 @ wattnsq_ref[...] + s1
    L3 = logitb.reshape(BN, K, C_NB)
    m = jnp.max(L3, axis=1, keepdims=True)
    e = jnp.exp(L3 - m)
    p3 = e * (1.0 / jnp.sum(e, axis=1, keepdims=True))

    nb3 = nb2.reshape(BN, K, C_NB)
    aggn_ref[...] = jnp.sum(p3 * nb3, axis=1)
    p_ref[...] = p3[:, :, 0]


def _tc_attn(xyz_pad, meta_g, W9, w9, bnb, ones3, wattnsq):
    nph = xyz_pad.shape[0]
    return pl.pallas_call(
        _tc_attn_body,
        grid=(nph // BN,),
        in_specs=[
            pl.BlockSpec((BN, 16), lambda i: (i, 0)),
            pl.BlockSpec((BN * K, 16), lambda i: (i, 0)),
            pl.BlockSpec((9, C_NB), lambda i: (0, 0)),
            pl.BlockSpec((1, C_NB), lambda i: (0, 0)),
            pl.BlockSpec((1, C_NB), lambda i: (0, 0)),
            pl.BlockSpec((3, C_NB), lambda i: (0, 0)),
            pl.BlockSpec((C_NB, C_NB), lambda i: (0, 0)),
        ],
        out_specs=[
            pl.BlockSpec((BN, K), lambda i: (i, 0)),
            pl.BlockSpec((BN, C_NB), lambda i: (i, 0)),
        ],
        out_shape=[
            jax.ShapeDtypeStruct((nph, K), jnp.float32),
            jax.ShapeDtypeStruct((nph, C_NB), jnp.float32),
        ],
    )(xyz_pad, meta_g, W9, w9, bnb, ones3, wattnsq)


def _tc_final_body(xyzp_ref, feat_ref, aggf_ref, aggn_ref,
                   Wof_ref, Won_ref, bout_ref, Wsc_ref, bsc_ref,
                   Wco_ref, bco_ref, fea_ref, pred_ref):
    out = (aggf_ref[...] @ Wof_ref[...] + aggn_ref[...] @ Won_ref[...]
           + bout_ref[...])
    shortcut = feat_ref[...] @ Wsc_ref[...] + bsc_ref[...]
    fea = out + shortcut
    fea = jnp.where(fea >= 0, fea, 0.2 * fea)
    pieces = [fea[:, u * C_IN:(u + 1) * C_IN] for u in range(UP)]
    fea_ref[...] = jnp.stack(pieces, axis=1).reshape(BN * UP, C_IN)

    xyzc = xyzp_ref[:, 0:3]
    Wco = Wco_ref[...]
    bco = bco_ref[...]
    offs = []
    for u in range(UP):
        off = pieces[u] @ Wco + bco
        offs.append(off + xyzc)
    pred_ref[...] = jnp.concatenate(offs, axis=1)


def _tc_final(xyz_pad, feat_pad, agg_f, agg_n, Wof, Won, bout, Wsc, bsc,
              Wco, bco):
    full = lambda shape: pl.BlockSpec(shape, lambda i: tuple(0 for _ in shape))
    return pl.pallas_call(
        _tc_final_body,
        grid=(GRID,),
        in_specs=[
            pl.BlockSpec((BN, 16), lambda i: (i, 0)),
            pl.BlockSpec((BN, C_IN), lambda i: (i, 0)),
            pl.BlockSpec((BN, C_IN), lambda i: (i, 0)),
            pl.BlockSpec((BN, C_NB), lambda i: (i, 0)),
            full((C_IN, C_OUT)),
            full((C_NB, C_OUT)),
            full((1, C_OUT)),
            full((C_IN, C_OUT)),
            full((1, C_OUT)),
            full((C_IN, 3)),
            full((1, 3)),
        ],
        out_specs=[
            pl.BlockSpec((BN * UP, C_IN), lambda i: (i, 0)),
            pl.BlockSpec((BN, 12), lambda i: (i, 0)),
        ],
        out_shape=[
            jax.ShapeDtypeStruct((NP * UP, C_IN), jnp.float32),
            jax.ShapeDtypeStruct((NP, 12), jnp.float32),
        ],
    )(xyz_pad, feat_pad, agg_f, agg_n, Wof, Won, bout, Wsc, bsc, Wco, bco)


def _sc_meta_gather(meta, idx2):
    return _sc_meta_gather_kernel()(meta, idx2)


def _sc_wreduce(feat_pad, idx2, p):
    return _sc_wreduce_kernel(p.shape[0])(feat_pad, idx2, p)


def kernel(xyz, feature, neighbors_idx, W_nb, b_nb, w_att, W_out, b_out,
           W_sc, b_sc, W_coord, b_coord):
    xyzf = xyz.reshape(N, 3)
    featf = feature.reshape(N, C_IN)
    idx = neighbors_idx.reshape(N * K).astype(jnp.int32)

    xyz_pad = jnp.pad(xyzf, ((0, NP - N), (0, 13)))
    feat_pad = jnp.pad(featf, ((0, NP - N), (0, 0)))
    idx2 = jnp.pad(idx, (0, NKP - N * K)).reshape(NKP // CH, CH)

    featbf = jnp.pad(featf.astype(jnp.bfloat16), ((0, NP - N), (0, 0)))
    featpk = lax.bitcast_convert_type(
        featbf.reshape(NP, C_IN // 2, 2), jnp.int32)

    wattf = w_att[:C_IN].reshape(1, C_IN)
    wattn = w_att[C_IN:].reshape(1, C_NB)
    perm = [32 * (m // 2) + 2 * t + (m % 2)
            for m in range(C_IN // 16) for t in range(16)]
    Wof = W_out[:C_IN][jnp.array(perm, jnp.int32)]
    Won = W_out[C_IN:]

    W9 = W_nb[0:9]
    w9 = W_nb[9:10]
    ones3 = jnp.ones((3, C_NB), jnp.float32)
    wattnsq = jnp.broadcast_to(w_att[C_IN:].reshape(C_NB, 1), (C_NB, C_NB))

    meta = _tc_meta(xyz_pad, feat_pad, wattf)
    meta_g = _sc_meta_gather(meta, idx2)

    p, agg_n = _tc_attn(xyz_pad, meta_g, W9, w9,
                        b_nb.reshape(1, C_NB), ones3, wattnsq)
    agg_f = _sc_wreduce(featpk, idx2, p)

    fea4, pred = _tc_final(
        xyz_pad, feat_pad, agg_f, agg_n,
        Wof, Won, b_out.reshape(1, C_OUT),
        W_sc, b_sc.reshape(1, C_OUT),
        W_coord, b_coord.reshape(1, 3),
    )

    pred_coord = pred[:N].reshape(1, N * UP, 3)
    fea_out = fea4[:N * UP].reshape(1, N * UP, C_IN)
    return pred_coord, fea_out
